# baseline (speedup 1.0000x reference)
import functools

import jax
import jax.numpy as jnp
from jax.experimental import pallas as pl

_MARGIN = 1.0
_BIG = 4096.0


def _triplet_block(a_ref, b_ref, ohl_ref, ohr_ref, out_ref):
    i = pl.program_id(0)

    t = jax.lax.dot_general(
        a_ref[...], b_ref[...],
        dimension_numbers=(((1,), (1,)), ((), ())),
        preferred_element_type=jnp.float32,
    )
    m = jax.lax.dot_general(
        ohl_ref[...], ohr_ref[...],
        dimension_numbers=(((1,), (1,)), ((), ())),
        preferred_element_type=jnp.float32,
    )
    u = t + m
    mx = jnp.max(u, axis=1)
    mn = jnp.min(u, axis=1)
    per = jax.nn.relu(mx - (_BIG - _MARGIN) - mn)
    partial = jnp.sum(per).reshape(1, 1)

    @pl.when(i == 0)
    def _init():
        out_ref[...] = jnp.zeros((1, 1), jnp.float32)

    out_ref[...] += partial


def kernel(x, y):
    n, d = x.shape
    br = 512
    grid = n // br

    x2 = jnp.sum(x * x, axis=1, keepdims=True)
    pad = jnp.zeros((n, 7), jnp.float32)
    a = jnp.concatenate([-2.0 * x, jnp.ones((n, 1), jnp.float32), pad], 1)
    b = jnp.concatenate([x, x2, pad], 1)
    oh = (y == jnp.arange(128, dtype=y.dtype)[None, :]).astype(jnp.bfloat16)
    ohr = oh * jnp.bfloat16(_BIG)

    out = pl.pallas_call(
        _triplet_block,
        grid=(grid,),
        in_specs=[
            pl.BlockSpec((br, 24), lambda i: (i, 0)),
            pl.BlockSpec((n, 24), lambda i: (0, 0)),
            pl.BlockSpec((br, 128), lambda i: (i, 0)),
            pl.BlockSpec((n, 128), lambda i: (0, 0)),
        ],
        out_specs=pl.BlockSpec((1, 1), lambda i: (0, 0)),
        out_shape=jax.ShapeDtypeStruct((1, 1), jnp.float32),
    )(a, b, oh, ohr)
    return out[0, 0] / n


# speedup vs baseline: 1.1473x; 1.1473x over previous
import functools

import jax
import jax.numpy as jnp
from jax.experimental import pallas as pl
from jax.experimental.pallas import tpu as pltpu

_MARGIN = 1.0
_BIG = 4096.0


def _triplet_block(x_ref, y_ref, out_ref, xs_ref, x2_ref, ohl_ref, ohr_ref,
                   *, br, n):
    i = pl.program_id(0)

    @pl.when(i == 0)
    def _prep():
        xv = x_ref[...]
        xs_ref[...] = -2.0 * xv
        x2_ref[...] = jax.lax.dot_general(
            jnp.ones((1, xv.shape[1]), jnp.float32), xv * xv,
            dimension_numbers=(((1,), (1,)), ((), ())),
            preferred_element_type=jnp.float32,
        )
        classes = jax.lax.broadcasted_iota(jnp.int32, (n, 128), 1)
        eq = y_ref[...] == classes
        eqf = jnp.where(eq, 1.0, 0.0)
        ohl_ref[...] = eqf.astype(jnp.bfloat16)
        ohr_ref[...] = (eqf * _BIG).astype(jnp.bfloat16)

    xb = xs_ref[pl.ds(i * br, br), :]
    g = jax.lax.dot_general(
        xb, x_ref[...],
        dimension_numbers=(((1,), (1,)), ((), ())),
        preferred_element_type=jnp.float32,
    )
    m = jax.lax.dot_general(
        ohl_ref[pl.ds(i * br, br), :], ohr_ref[...],
        dimension_numbers=(((1,), (1,)), ((), ())),
        preferred_element_type=jnp.float32,
    )
    u = (g + m) + x2_ref[...]
    mx = jnp.max(u, axis=1)
    mn = jnp.min(u, axis=1)
    per = jax.nn.relu(mx - (_BIG - _MARGIN) - mn)
    partial = jnp.sum(per).reshape(1, 1)

    @pl.when(i == 0)
    def _init():
        out_ref[...] = jnp.zeros((1, 1), jnp.float32)

    out_ref[...] += partial


def kernel(x, y):
    n, d = x.shape
    br = 512
    grid = n // br

    out = pl.pallas_call(
        functools.partial(_triplet_block, br=br, n=n),
        grid=(grid,),
        in_specs=[
            pl.BlockSpec((n, d), lambda i: (0, 0)),
            pl.BlockSpec((n, 1), lambda i: (0, 0)),
        ],
        out_specs=pl.BlockSpec((1, 1), lambda i: (0, 0)),
        out_shape=jax.ShapeDtypeStruct((1, 1), jnp.float32),
        scratch_shapes=[
            pltpu.VMEM((n, d), jnp.float32),
            pltpu.VMEM((1, n), jnp.float32),
            pltpu.VMEM((n, 128), jnp.bfloat16),
            pltpu.VMEM((n, 128), jnp.bfloat16),
        ],
    )(x, y.reshape(n, 1))
    return out[0, 0] / n


# speedup vs baseline: 1.1695x; 1.0193x over previous
import functools

import jax
import jax.numpy as jnp
from jax.experimental import pallas as pl
from jax.experimental.pallas import tpu as pltpu

_MARGIN = 1.0
_BIG = 4096.0


def _triplet_block(x_ref, y_ref, out_ref, xs_ref, ohl_ref, ohr_ref, *, br, n):
    i = pl.program_id(0)

    @pl.when(i == 0)
    def _prep():
        xv = x_ref[...]
        xs_ref[...] = -2.0 * xv
        x2 = jnp.sum(xv * xv, axis=1, keepdims=True)
        x2h = x2.astype(jnp.bfloat16).astype(jnp.float32)
        x2l = x2 - x2h
        classes = jax.lax.broadcasted_iota(jnp.int32, (n, 128), 1)
        eqf = jnp.where(y_ref[...] == classes, 1.0, 0.0)
        is_h = (classes == 100).astype(jnp.float32)
        is_l = (classes == 101).astype(jnp.float32)
        ohl_ref[...] = (eqf + is_h + is_l).astype(jnp.bfloat16)
        ohr_ref[...] = (eqf * _BIG + is_h * x2h + is_l * x2l
                        ).astype(jnp.bfloat16)

    xb = xs_ref[pl.ds(i * br, br), :]
    g = jax.lax.dot_general(
        xb, x_ref[...],
        dimension_numbers=(((1,), (1,)), ((), ())),
        preferred_element_type=jnp.float32,
    )
    m = jax.lax.dot_general(
        ohl_ref[pl.ds(i * br, br), :], ohr_ref[...],
        dimension_numbers=(((1,), (1,)), ((), ())),
        preferred_element_type=jnp.float32,
    )
    u = g + m
    mx = jnp.max(u, axis=1)
    mn = jnp.min(u, axis=1)
    per = jax.nn.relu(mx - (_BIG - _MARGIN) - mn)
    partial = jnp.sum(per).reshape(1, 1)

    @pl.when(i == 0)
    def _init():
        out_ref[...] = jnp.zeros((1, 1), jnp.float32)

    out_ref[...] += partial


def kernel(x, y):
    n, d = x.shape
    br = 512
    grid = n // br

    out = pl.pallas_call(
        functools.partial(_triplet_block, br=br, n=n),
        grid=(grid,),
        in_specs=[
            pl.BlockSpec((n, d), lambda i: (0, 0)),
            pl.BlockSpec((n, 1), lambda i: (0, 0)),
        ],
        out_specs=pl.BlockSpec((1, 1), lambda i: (0, 0)),
        out_shape=jax.ShapeDtypeStruct((1, 1), jnp.float32),
        scratch_shapes=[
            pltpu.VMEM((n, d), jnp.float32),
            pltpu.VMEM((n, 128), jnp.bfloat16),
            pltpu.VMEM((n, 128), jnp.bfloat16),
        ],
    )(x, y.reshape(n, 1))
    return out[0, 0] / n


# speedup vs baseline: 1.2221x; 1.0450x over previous
import functools

import jax
import jax.numpy as jnp
from jax.experimental import pallas as pl
from jax.experimental.pallas import tpu as pltpu

_MARGIN = 1.0
_BIG = 4096.0


def _triplet_block(x_ref, y_ref, out_ref, xs_ref, ohl_ref, ohr_ref, *, br, n):
    i = pl.program_id(0)

    @pl.when(i == 0)
    def _prep():
        xv = x_ref[...]
        xs_ref[...] = -2.0 * xv
        x2 = jnp.sum(xv * xv, axis=1, keepdims=True)
        x2h = x2.astype(jnp.bfloat16).astype(jnp.float32)
        x2l = x2 - x2h
        classes = jax.lax.broadcasted_iota(jnp.int32, (n, 128), 1)
        eqf = jnp.where(y_ref[...] == classes, 1.0, 0.0)
        is_h = (classes == 100).astype(jnp.float32)
        is_l = (classes == 101).astype(jnp.float32)
        ohl_ref[...] = (eqf + is_h + is_l).astype(jnp.bfloat16)
        ohr_ref[...] = (eqf * _BIG + is_h * x2h + is_l * x2l
                        ).astype(jnp.bfloat16)

    xb = xs_ref[pl.ds(i * br, br), :]
    g = jax.lax.dot_general(
        xb, x_ref[...],
        dimension_numbers=(((1,), (1,)), ((), ())),
        preferred_element_type=jnp.float32,
    )
    m = jax.lax.dot_general(
        ohl_ref[pl.ds(i * br, br), :], ohr_ref[...],
        dimension_numbers=(((1,), (1,)), ((), ())),
        preferred_element_type=jnp.float32,
    )
    u = g + m
    mx = jnp.max(u, axis=1)
    mn = jnp.min(u, axis=1)
    per = jax.nn.relu(mx - (_BIG - _MARGIN) - mn)
    partial = jnp.sum(per).reshape(1, 1)

    @pl.when(i == 0)
    def _init():
        out_ref[...] = jnp.zeros((1, 1), jnp.float32)

    out_ref[...] += partial


def kernel(x, y):
    n, d = x.shape
    br = 1024
    grid = n // br

    out = pl.pallas_call(
        functools.partial(_triplet_block, br=br, n=n),
        grid=(grid,),
        in_specs=[
            pl.BlockSpec((n, d), lambda i: (0, 0)),
            pl.BlockSpec((n, 1), lambda i: (0, 0)),
        ],
        out_specs=pl.BlockSpec((1, 1), lambda i: (0, 0)),
        out_shape=jax.ShapeDtypeStruct((1, 1), jnp.float32),
        scratch_shapes=[
            pltpu.VMEM((n, d), jnp.float32),
            pltpu.VMEM((n, 128), jnp.bfloat16),
            pltpu.VMEM((n, 128), jnp.bfloat16),
        ],
    )(x, y.reshape(n, 1))
    return out[0, 0] / n


# speedup vs baseline: 1.6338x; 1.3369x over previous
import functools

import jax
import jax.numpy as jnp
from jax.experimental import pallas as pl
from jax.experimental.pallas import tpu as pltpu

_MARGIN = 1.0
_BIG = 4096.0


def _triplet_block(x_ref, y_ref, out_ref, lhs_ref, rhs_ref, *, br, n):
    i = pl.program_id(0)

    @pl.when(i == 0)
    def _prep():
        xv = x_ref[...]
        xh = xv.astype(jnp.bfloat16)
        xl = (xv - xh.astype(jnp.float32)).astype(jnp.bfloat16)
        x2 = jnp.sum(xv * xv, axis=1, keepdims=True)
        x2h = x2.astype(jnp.bfloat16).astype(jnp.float32)
        x2l = x2 - x2h
        classes = jax.lax.broadcasted_iota(jnp.int32, (n, 128), 1)
        eqf = jnp.where(y_ref[...] == classes, 1.0, 0.0)
        is_h = (classes == 100).astype(jnp.float32)
        is_l = (classes == 101).astype(jnp.float32)
        ohl = (eqf + is_h + is_l).astype(jnp.bfloat16)
        ohr = (eqf * _BIG + is_h * x2h + is_l * x2l).astype(jnp.bfloat16)
        mh = jnp.bfloat16(-2.0) * xh
        ml = jnp.bfloat16(-2.0) * xl
        lhs_ref[...] = jnp.concatenate([mh, mh, ml, ohl], axis=1)
        rhs_ref[...] = jnp.concatenate([xh, xl, xh, ohr], axis=1)

    u = jax.lax.dot_general(
        lhs_ref[pl.ds(i * br, br), :], rhs_ref[...],
        dimension_numbers=(((1,), (1,)), ((), ())),
        preferred_element_type=jnp.float32,
    )
    mx = jnp.max(u, axis=1)
    mn = jnp.min(u, axis=1)
    per = jax.nn.relu(mx - (_BIG - _MARGIN) - mn)
    partial = jnp.sum(per).reshape(1, 1)

    @pl.when(i == 0)
    def _init():
        out_ref[...] = jnp.zeros((1, 1), jnp.float32)

    out_ref[...] += partial


def kernel(x, y):
    n, d = x.shape
    br = 1024
    grid = n // br

    out = pl.pallas_call(
        functools.partial(_triplet_block, br=br, n=n),
        grid=(grid,),
        in_specs=[
            pl.BlockSpec((n, d), lambda i: (0, 0)),
            pl.BlockSpec((n, 1), lambda i: (0, 0)),
        ],
        out_specs=pl.BlockSpec((1, 1), lambda i: (0, 0)),
        out_shape=jax.ShapeDtypeStruct((1, 1), jnp.float32),
        scratch_shapes=[
            pltpu.VMEM((n, 3 * d + 128), jnp.bfloat16),
            pltpu.VMEM((n, 3 * d + 128), jnp.bfloat16),
        ],
    )(x, y.reshape(n, 1))
    return out[0, 0] / n


# speedup vs baseline: 1.6900x; 1.0344x over previous
import functools

import jax
import jax.numpy as jnp
from jax.experimental import pallas as pl
from jax.experimental.pallas import tpu as pltpu

_MARGIN = 1.0
_BIG = 4096.0


def _triplet_block(x_ref, y_ref, out_ref, lhs_ref, rhs_ref, *, br, n):
    i = pl.program_id(0)

    @pl.when(i == 0)
    def _prep():
        xv = x_ref[...]
        xh = xv.astype(jnp.bfloat16)
        xl = (xv - xh.astype(jnp.float32)).astype(jnp.bfloat16)
        x2 = jnp.sum(xv * xv, axis=1, keepdims=True)
        x2h = x2.astype(jnp.bfloat16).astype(jnp.float32)
        x2l = x2 - x2h
        classes = jax.lax.broadcasted_iota(jnp.int32, (n, 128), 1)
        eqf = jnp.where(y_ref[...] == classes, 1.0, 0.0)
        is_h = (classes == 100).astype(jnp.float32)
        is_l = (classes == 101).astype(jnp.float32)
        ohl = (eqf + is_h + is_l).astype(jnp.bfloat16)
        ohr = (eqf * _BIG + is_h * x2h + is_l * x2l).astype(jnp.bfloat16)
        mh = jnp.bfloat16(-2.0) * xh
        ml = jnp.bfloat16(-2.0) * xl
        lhs_ref[...] = jnp.concatenate([mh, mh, ml, ohl], axis=1)
        rhs_ref[...] = jnp.concatenate([xh, xl, xh, ohr], axis=1)

    u = jax.lax.dot_general(
        lhs_ref[pl.ds(i * br, br), :], rhs_ref[...],
        dimension_numbers=(((1,), (1,)), ((), ())),
        preferred_element_type=jnp.float32,
    )
    mx = jnp.max(u, axis=1)
    mn = jnp.min(u, axis=1)
    per = jax.nn.relu(mx - (_BIG - _MARGIN) - mn)
    partial = jnp.sum(per).reshape(1, 1)

    @pl.when(i == 0)
    def _init():
        out_ref[...] = jnp.zeros((1, 1), jnp.float32)

    out_ref[...] += partial


def kernel(x, y):
    n, d = x.shape
    br = 2048
    grid = n // br

    out = pl.pallas_call(
        functools.partial(_triplet_block, br=br, n=n),
        grid=(grid,),
        in_specs=[
            pl.BlockSpec((n, d), lambda i: (0, 0)),
            pl.BlockSpec((n, 1), lambda i: (0, 0)),
        ],
        out_specs=pl.BlockSpec((1, 1), lambda i: (0, 0)),
        out_shape=jax.ShapeDtypeStruct((1, 1), jnp.float32),
        scratch_shapes=[
            pltpu.VMEM((n, 3 * d + 128), jnp.bfloat16),
            pltpu.VMEM((n, 3 * d + 128), jnp.bfloat16),
        ],
    )(x, y.reshape(n, 1))
    return out[0, 0] / n
